# baseline (speedup 1.0000x reference)
import jax
import jax.numpy as jnp
from jax import lax
from jax.experimental import pallas as pl

B, D1, D2 = 4, 4096, 2048
CB = 256
RV = D1 // 8


def _argmin_tc(x):
    ncb = D2 // CB

    def body(x_ref, o_ref):
        iota_r = lax.broadcasted_iota(jnp.int32, (RV, 8, CB), 0)
        iota_s = lax.broadcasted_iota(jnp.int32, (8, CB), 0)
        xr = x_ref[0].reshape(RV, 8, CB)
        gmin = jnp.min(xr, axis=(0, 1), keepdims=True)
        cand = jnp.where(xr == gmin, iota_r, jnp.int32(RV))
        cr = jnp.min(cand, axis=0)
        rows = cr * 8 + iota_s
        o_ref[0, 0, 0, :] = jnp.min(rows, axis=0)

    out = pl.pallas_call(
        body,
        grid=(ncb, B),
        in_specs=[pl.BlockSpec((1, D1, CB), lambda c, b: (b, 0, c))],
        out_specs=pl.BlockSpec((1, 1, 1, CB), lambda c, b: (b, c, 0, 0)),
        out_shape=jax.ShapeDtypeStruct((B, ncb, 1, CB), jnp.int32),
    )(x)
    return out.reshape(B, D2)


def kernel(x):
    return _argmin_tc(x)


# speedup vs baseline: 1.2668x; 1.2668x over previous
import jax
import jax.numpy as jnp
from jax import lax
from jax.experimental import pallas as pl

B, D1, D2 = 4, 4096, 2048
CB = 256
RV = D1 // 8


def _argmin_tc(x):
    ncb = D2 // CB

    def body(x_ref, o_ref):
        iota_r = lax.broadcasted_iota(jnp.int32, (RV, 8, CB), 0)
        iota_s = lax.broadcasted_iota(jnp.int32, (8, CB), 0)
        for b in range(B):
            xr = x_ref[b].reshape(RV, 8, CB)
            gmin = jnp.min(xr, axis=(0, 1), keepdims=True)
            cand = jnp.where(xr == gmin, iota_r, jnp.int32(RV))
            cr = jnp.min(cand, axis=0)
            rows = cr * 8 + iota_s
            o_ref[b, :] = jnp.min(rows, axis=0)

    return pl.pallas_call(
        body,
        grid=(ncb,),
        in_specs=[pl.BlockSpec((B, D1, CB), lambda c: (0, 0, c))],
        out_specs=pl.BlockSpec((B, CB), lambda c: (0, c)),
        out_shape=jax.ShapeDtypeStruct((B, D2), jnp.int32),
    )(x)


def kernel(x):
    return _argmin_tc(x)
